# baseline (speedup 1.0000x reference)
import functools

import jax
import jax.numpy as jnp
from jax import lax
from jax.experimental import pallas as pl
from jax.experimental.pallas import tpu as pltpu
from jax.experimental.pallas import tpu_sc as plsc

ROWS = 16384
SEQ = 50
DIM = 64
NC, NS = 2, 16
NW = NC * NS
RPW = ROWS // NW
NBUF = 8
LAG = NBUF // 2


def _emb_body(idx_hbm, table_hbm, out_hbm, idx_v, rows, sg, so):
    wid = lax.axis_index("s") * NC + lax.axis_index("c")
    base = wid * RPW
    pltpu.sync_copy(idx_hbm.at[pl.ds(base, RPW)], idx_v)

    def gather(t, b):
        pltpu.async_copy(table_hbm.at[idx_v.at[t]], rows[b], sg[b])

    def gather_wait(t, b):
        pltpu.make_async_copy(table_hbm.at[idx_v.at[t]], rows[b], sg[b]).wait()

    def write_out(t, b):
        pltpu.async_copy(rows[b], out_hbm.at[base + t], so[b])

    def write_wait(t, b):
        pltpu.make_async_copy(rows[b], out_hbm.at[base + t], so[b]).wait()

    for i in range(NBUF):
        gather(i, i)
        if i >= LAG:
            j = i - LAG
            gather_wait(j, j)
            write_out(j, j)

    def group(g, _):
        for b in range(NBUF):
            i = NBUF * g + b
            j = i - LAG
            bj = (b - LAG) % NBUF
            write_wait(i - NBUF, b)
            gather(i, b)
            gather_wait(j, bj)
            write_out(j, bj)
        return _

    lax.fori_loop(1, RPW // NBUF, group, None)

    for j in range(RPW - LAG, RPW):
        bj = j % NBUF
        gather_wait(j, bj)
        write_out(j, bj)
    for j in range(RPW - NBUF, RPW):
        write_wait(j, j % NBUF)


@jax.jit
def _embedding_lookup(idx, weight):
    mesh = plsc.VectorSubcoreMesh(core_axis_name="c", subcore_axis_name="s")
    k = functools.partial(
        pl.kernel,
        out_type=jax.ShapeDtypeStruct((ROWS, SEQ, DIM), jnp.float32),
        mesh=mesh,
        scratch_types=[
            pltpu.VMEM((RPW, SEQ), jnp.int32),
            [pltpu.VMEM((SEQ, DIM), jnp.float32) for _ in range(NBUF)],
            [pltpu.SemaphoreType.DMA for _ in range(NBUF)],
            [pltpu.SemaphoreType.DMA for _ in range(NBUF)],
        ],
        compiler_params=pltpu.CompilerParams(use_tc_tiling_on_sc=False),
    )(_emb_body)
    return k(idx, weight)


def kernel(token_ids, weight):
    return _embedding_lookup(token_ids.astype(jnp.int32), weight)
